# baseline (speedup 1.0000x reference)
import functools

import jax
import jax.numpy as jnp
from jax import lax
from jax.experimental import pallas as pl
from jax.experimental.pallas import tpu as pltpu
from jax.experimental.pallas import tpu_sc as plsc

V = 100000
D = 64
L = 200
LP = 256
LR = 208
B = 4096

NC = 2
NS = 16
NW = NC * NS
BPW = B // NW
VBLK = 2000

NCH = 4
CB = BPW // NCH


def _mm_body(emb_ref, wt_ref, out_ref):
    res = jnp.dot(emb_ref[...], wt_ref[...],
                  preferred_element_type=jnp.float32)
    out_ref[...] = res.reshape(2 * VBLK, 128)


def _scores_matmul(embedding, wt):
    return pl.pallas_call(
        _mm_body,
        grid=(V // VBLK,),
        in_specs=[
            pl.BlockSpec((VBLK, D), lambda i: (i, 0)),
            pl.BlockSpec((D, LP), lambda i: (0, 0)),
        ],
        out_specs=pl.BlockSpec((2 * VBLK, 128), lambda i: (i, 0)),
        out_shape=jax.ShapeDtypeStruct((2 * V, 128), jnp.float32),
    )(embedding, wt)


@functools.cache
def _make_sc_gather_reduce():
  @functools.partial(
      pl.kernel,
      out_type=jax.ShapeDtypeStruct((B,), jnp.float32),
      mesh=plsc.VectorSubcoreMesh(core_axis_name="c", subcore_axis_name="s",
                                  num_cores=NC, num_subcores=NS),
      scratch_types=[
          pltpu.VMEM((BPW, L), jnp.int32),
          pltpu.VMEM((BPW * LR,), jnp.int32),
          pltpu.VMEM((BPW * LR,), jnp.float32),
          pltpu.VMEM((BPW,), jnp.float32),
          pltpu.VMEM((BPW,), jnp.float32),
          pltpu.VMEM((16,), jnp.float32),
          [pltpu.SemaphoreType.DMA] * NCH,
      ],
  )
  def _sc_gather_reduce(x_hbm, scores_hbm, fcb_hbm, out_hbm,
                        xv, idx, vals, sums, outv, fcbv, sems):
    wid = lax.axis_index("s") * NC + lax.axis_index("c")
    base = wid * BPW

    pltpu.sync_copy(x_hbm.at[pl.ds(base, BPW), :], xv)
    pltpu.sync_copy(fcb_hbm, fcbv)

    lanes = lax.iota(jnp.int32, 16)
    tail_off = 184 + lanes + jnp.where(lanes < 8, 16, 0)

    copies = []
    for c in range(NCH):
      def build_b(b, carry):
        rb = b * LR
        for j in range(12):
          xj = xv[b, pl.ds(j * 16, 16)]
          idx[pl.ds(rb + j * 16, 16)] = xj * LP + (j * 16 + lanes)
        v8 = xv[b, pl.ds(L - 16, 16)]
        idx[pl.ds(rb + 192, 16)] = v8 * LP + tail_off
        return carry
      lax.fori_loop(c * CB, (c + 1) * CB, build_b, 0)
      sl = pl.ds(c * CB * LR, CB * LR)
      copies.append(
          pltpu.async_copy(scores_hbm.at[idx.at[sl]], vals.at[sl], sems[c]))

    for c in range(NCH):
      copies[c].wait()
      def reduce_g(g, carry):
        out_vec = jnp.zeros((16,), jnp.float32)
        for i in range(16):
          rb = (g * 16 + i) * LR
          s = vals[pl.ds(rb, 16)]
          for j in range(1, 13):
            s = s + vals[pl.ds(rb + j * 16, 16)]
          for k in (1, 2, 4, 8):
            s = s + s[lanes ^ k]
          out_vec = jnp.where(lanes == i, s, out_vec)
        sums[pl.ds(g * 16, 16)] = out_vec
        return carry
      lax.fori_loop(c * (CB // 16), (c + 1) * (CB // 16), reduce_g, 0)

    bias = fcbv[...]
    for k in range(BPW // 16):
      z = sums[pl.ds(k * 16, 16)] + bias
      outv[pl.ds(k * 16, 16)] = 1.0 / (1.0 + jnp.exp(-z))

    pltpu.sync_copy(outv, out_hbm.at[pl.ds(base, BPW)])

  return _sc_gather_reduce


def kernel(x, embedding, fc_w, fc_b):
    x = x.astype(jnp.int32)
    wt = jnp.pad(fc_w[:, 0].reshape(L, D).T, ((0, 0), (0, LP - L)))
    scores = _scores_matmul(embedding, wt)
    scores_flat = scores.reshape(V * LP)
    fcb16 = jnp.broadcast_to(fc_b[0], (16,)).astype(jnp.float32)
    out = _make_sc_gather_reduce()(x, scores_flat, fcb16)
    return out.reshape(B, 1)


# speedup vs baseline: 1.0155x; 1.0155x over previous
import functools

import jax
import jax.numpy as jnp
from jax import lax
from jax.experimental import pallas as pl
from jax.experimental.pallas import tpu as pltpu
from jax.experimental.pallas import tpu_sc as plsc

V = 100000
D = 64
L = 200
LP = 256
LR = 208
B = 4096

NC = 2
NS = 16
NW = NC * NS
BPW = B // NW
VBLK = 2000

NCH = 4
CB = BPW // NCH


def _round_bf16_bits(r):
    u = jax.lax.bitcast_convert_type(r, jnp.uint32)
    return ((u + 0x7FFF + ((u >> 16) & 1)) >> 16) & 0xFFFF


def _mm_body(emb_ref, wte_ref, wto_ref, out_ref):
    re = jnp.dot(emb_ref[...], wte_ref[...], preferred_element_type=jnp.float32)
    ro = jnp.dot(emb_ref[...], wto_ref[...], preferred_element_type=jnp.float32)
    packed = _round_bf16_bits(re) | (_round_bf16_bits(ro) << 16)
    out_ref[...] = jax.lax.bitcast_convert_type(packed, jnp.int32)


def _scores_matmul(embedding, wte, wto):
    return pl.pallas_call(
        _mm_body,
        grid=(V // VBLK,),
        in_specs=[
            pl.BlockSpec((VBLK, D), lambda i: (i, 0)),
            pl.BlockSpec((D, LP // 2), lambda i: (0, 0)),
            pl.BlockSpec((D, LP // 2), lambda i: (0, 0)),
        ],
        out_specs=pl.BlockSpec((VBLK, LP // 2), lambda i: (i, 0)),
        out_shape=jax.ShapeDtypeStruct((V, LP // 2), jnp.int32),
    )(embedding, wte, wto)


@functools.cache
def _make_sc_gather_reduce():
  @functools.partial(
      pl.kernel,
      out_type=jax.ShapeDtypeStruct((B,), jnp.float32),
      mesh=plsc.VectorSubcoreMesh(core_axis_name="c", subcore_axis_name="s",
                                  num_cores=NC, num_subcores=NS),
      scratch_types=[
          pltpu.VMEM((BPW, L), jnp.int32),
          pltpu.VMEM((BPW * LR,), jnp.int32),
          pltpu.VMEM((BPW * LR,), jnp.int32),
          pltpu.VMEM((BPW,), jnp.float32),
          pltpu.VMEM((BPW,), jnp.float32),
          pltpu.VMEM((16,), jnp.float32),
          [pltpu.SemaphoreType.DMA] * NCH,
      ],
  )
  def _sc_gather_reduce(x_hbm, scores_hbm, fcb_hbm, out_hbm,
                        xv, idx, vals, sums, outv, fcbv, sems):
    wid = lax.axis_index("s") * NC + lax.axis_index("c")
    base = wid * BPW

    pltpu.sync_copy(x_hbm.at[pl.ds(base, BPW), :], xv)
    pltpu.sync_copy(fcb_hbm, fcbv)

    lanes = lax.iota(jnp.int32, 16)
    tail_off = 184 + lanes + jnp.where(lanes < 8, 16, 0)
    half = lanes >> 1
    tail_half = tail_off >> 1
    even = (lanes & 1) == 0
    himask = jnp.full((16,), -65536, jnp.int32)

    copies = []
    for c in range(NCH):
      def build_b(b, carry):
        rb = b * LR
        for j in range(12):
          xj = xv[b, pl.ds(j * 16, 16)]
          idx[pl.ds(rb + j * 16, 16)] = xj * (LP // 2) + (j * 8 + half)
        v8 = xv[b, pl.ds(L - 16, 16)]
        idx[pl.ds(rb + 192, 16)] = v8 * (LP // 2) + tail_half
        return carry
      lax.fori_loop(c * CB, (c + 1) * CB, build_b, 0)
      sl = pl.ds(c * CB * LR, CB * LR)
      copies.append(
          pltpu.async_copy(scores_hbm.at[idx.at[sl]], vals.at[sl], sems[c]))

    for c in range(NCH):
      copies[c].wait()
      def unpack(w):
        return lax.bitcast_convert_type(
            jnp.where(even, w << 16, w & himask), jnp.float32)

      def reduce_g(g, carry):
        out_vec = jnp.zeros((16,), jnp.float32)
        for i in range(16):
          rb = (g * 16 + i) * LR
          s = unpack(vals[pl.ds(rb, 16)])
          for j in range(1, 13):
            s = s + unpack(vals[pl.ds(rb + j * 16, 16)])
          for k in (1, 2, 4, 8):
            s = s + s[lanes ^ k]
          out_vec = jnp.where(lanes == i, s, out_vec)
        sums[pl.ds(g * 16, 16)] = out_vec
        return carry
      lax.fori_loop(c * (CB // 16), (c + 1) * (CB // 16), reduce_g, 0)

    bias = fcbv[...]
    for k in range(BPW // 16):
      z = sums[pl.ds(k * 16, 16)] + bias
      outv[pl.ds(k * 16, 16)] = 1.0 / (1.0 + jnp.exp(-z))

    pltpu.sync_copy(outv, out_hbm.at[pl.ds(base, BPW)])

  return _sc_gather_reduce


def kernel(x, embedding, fc_w, fc_b):
    x = x.astype(jnp.int32)
    wt = jnp.pad(fc_w[:, 0].reshape(L, D).T, ((0, 0), (0, LP - L)))
    scores = _scores_matmul(embedding, wt[:, 0::2], wt[:, 1::2])
    scores_flat = scores.reshape(V * (LP // 2))
    fcb16 = jnp.broadcast_to(fc_b[0], (16,)).astype(jnp.float32)
    out = _make_sc_gather_reduce()(x, scores_flat, fcb16)
    return out.reshape(B, 1)


# speedup vs baseline: 1.1514x; 1.1339x over previous
import functools

import jax
import jax.numpy as jnp
from jax import lax
from jax.experimental import pallas as pl
from jax.experimental.pallas import tpu as pltpu
from jax.experimental.pallas import tpu_sc as plsc

V = 100000
D = 64
L = 200
LP = 256
LR = 208
B = 4096

NC = 2
NS = 16
NW = NC * NS
BPW = B // NW
VBLK = 4000

NCH = 4
CB = BPW // NCH


def _mm_body(emb_ref, wte_ref, wto_ref, out_ref):
    re = jnp.dot(emb_ref[...], wte_ref[...], preferred_element_type=jnp.float32)
    ro = jnp.dot(emb_ref[...], wto_ref[...], preferred_element_type=jnp.float32)
    ue = jax.lax.bitcast_convert_type(re, jnp.uint32)
    uo = jax.lax.bitcast_convert_type(ro, jnp.uint32)
    packed = (ue >> 16) | (uo & jnp.uint32(0xFFFF0000))
    out_ref[...] = jax.lax.bitcast_convert_type(packed, jnp.int32)


def _scores_matmul(embedding, wte, wto):
    return pl.pallas_call(
        _mm_body,
        grid=(V // VBLK,),
        in_specs=[
            pl.BlockSpec((VBLK, D), lambda i: (i, 0)),
            pl.BlockSpec((D, LP // 2), lambda i: (0, 0)),
            pl.BlockSpec((D, LP // 2), lambda i: (0, 0)),
        ],
        out_specs=pl.BlockSpec((VBLK, LP // 2), lambda i: (i, 0)),
        out_shape=jax.ShapeDtypeStruct((V, LP // 2), jnp.int32),
    )(embedding, wte, wto)


@functools.cache
def _make_sc_gather_reduce():
  @functools.partial(
      pl.kernel,
      out_type=jax.ShapeDtypeStruct((B,), jnp.float32),
      mesh=plsc.VectorSubcoreMesh(core_axis_name="c", subcore_axis_name="s",
                                  num_cores=NC, num_subcores=NS),
      scratch_types=[
          pltpu.VMEM((BPW, L), jnp.int32),
          pltpu.VMEM((BPW * LR,), jnp.int32),
          pltpu.VMEM((BPW * LR,), jnp.int32),
          pltpu.VMEM((BPW,), jnp.float32),
          pltpu.VMEM((BPW,), jnp.float32),
          pltpu.VMEM((16,), jnp.float32),
          [pltpu.SemaphoreType.DMA] * NCH,
      ],
  )
  def _sc_gather_reduce(x_hbm, scores_hbm, fcb_hbm, out_hbm,
                        xv, idx, vals, sums, outv, fcbv, sems):
    wid = lax.axis_index("s") * NC + lax.axis_index("c")
    base = wid * BPW

    pltpu.sync_copy(x_hbm.at[pl.ds(base, BPW), :], xv)
    pltpu.sync_copy(fcb_hbm, fcbv)

    lanes = lax.iota(jnp.int32, 16)
    tail_off = 184 + lanes + jnp.where(lanes < 8, 16, 0)
    half = lanes >> 1
    tail_half = tail_off >> 1
    even = (lanes & 1) == 0
    himask = jnp.full((16,), -65536, jnp.int32)

    copies = []
    for c in range(NCH):
      def build_b(b, carry):
        rb = b * LR
        for j in range(12):
          xj = xv[b, pl.ds(j * 16, 16)]
          idx[pl.ds(rb + j * 16, 16)] = xj * (LP // 2) + (j * 8 + half)
        v8 = xv[b, pl.ds(L - 16, 16)]
        idx[pl.ds(rb + 192, 16)] = v8 * (LP // 2) + tail_half
        return carry
      lax.fori_loop(c * CB, (c + 1) * CB, build_b, 0)
      sl = pl.ds(c * CB * LR, CB * LR)
      copies.append(
          pltpu.async_copy(scores_hbm.at[idx.at[sl]], vals.at[sl], sems[c]))

    for c in range(NCH):
      copies[c].wait()
      def unpack(w):
        return lax.bitcast_convert_type(
            jnp.where(even, w << 16, w & himask), jnp.float32)

      def reduce_g(g, carry):
        out_vec = jnp.zeros((16,), jnp.float32)
        for i in range(16):
          rb = (g * 16 + i) * LR
          s = unpack(vals[pl.ds(rb, 16)])
          for j in range(1, 13):
            s = s + unpack(vals[pl.ds(rb + j * 16, 16)])
          for k in (1, 2, 4, 8):
            s = s + s[lanes ^ k]
          out_vec = jnp.where(lanes == i, s, out_vec)
        sums[pl.ds(g * 16, 16)] = out_vec
        return carry
      lax.fori_loop(c * (CB // 16), (c + 1) * (CB // 16), reduce_g, 0)

    bias = fcbv[...]
    for k in range(BPW // 16):
      z = sums[pl.ds(k * 16, 16)] + bias
      outv[pl.ds(k * 16, 16)] = 1.0 / (1.0 + jnp.exp(-z))

    pltpu.sync_copy(outv, out_hbm.at[pl.ds(base, BPW)])

  return _sc_gather_reduce


def kernel(x, embedding, fc_w, fc_b):
    x = x.astype(jnp.int32)
    wt = jnp.pad(fc_w[:, 0].reshape(L, D).T, ((0, 0), (0, LP - L)))
    scores = _scores_matmul(embedding, wt[:, 0::2], wt[:, 1::2])
    scores_flat = scores.reshape(V * (LP // 2))
    fcb16 = jnp.broadcast_to(fc_b[0], (16,)).astype(jnp.float32)
    out = _make_sc_gather_reduce()(x, scores_flat, fcb16)
    return out.reshape(B, 1)
